# initial kernel scaffold (unmeasured)
import jax
import jax.numpy as jnp
from jax import lax
from jax.experimental import pallas as pl
from jax.experimental.pallas import tpu as pltpu

T = 1024
D = 1024
F = 2048
E_LOC = 2

_MESH = pl.DeviceIdType.MESH


def kernel(x, assign, W1, W2):
    assign2 = assign.reshape(T, 1)

    def body(x_ref, a_ref, w1_ref, w2_ref, out_ref,
             xb_ref, xpeer_ref, apeer_ref, contrib_ref, recvc_ref,
             send_sems, recv_sems):
        my_x = lax.axis_index("x")
        my_y = lax.axis_index("y")
        my_z = lax.axis_index("z")
        peer = (1 - my_x, my_y, my_z)

        barrier = pltpu.get_barrier_semaphore()
        pl.semaphore_signal(barrier, inc=1, device_id=peer,
                            device_id_type=_MESH)
        pl.semaphore_wait(barrier, 1)

        xb_ref[...] = x_ref[...].astype(jnp.bfloat16)

        rdma_x = pltpu.make_async_remote_copy(
            src_ref=xb_ref, dst_ref=xpeer_ref,
            send_sem=send_sems.at[0], recv_sem=recv_sems.at[0],
            device_id=peer, device_id_type=_MESH)
        rdma_x.start()
        rdma_a = pltpu.make_async_remote_copy(
            src_ref=a_ref, dst_ref=apeer_ref,
            send_sem=send_sems.at[1], recv_sem=recv_sems.at[1],
            device_id=peer, device_id_type=_MESH)
        rdma_a.start()

        def moe(xv, av):
            acc = None
            for k in range(E_LOC):
                ge = E_LOC * my_x + k
                xm = jnp.where(av == ge, xv, jnp.bfloat16(0.0))
                h = jnp.dot(xm, w1_ref[k].astype(jnp.bfloat16),
                            preferred_element_type=jnp.float32)
                h = jnp.maximum(h, 0.0).astype(jnp.bfloat16)
                o = jnp.dot(h, w2_ref[k].astype(jnp.bfloat16),
                            preferred_element_type=jnp.float32)
                acc = o if acc is None else acc + o
            return acc

        out_ref[...] = moe(xb_ref[...], a_ref[...])

        rdma_x.wait()
        rdma_a.wait()
        contrib_ref[...] = moe(xpeer_ref[...], apeer_ref[...]).astype(
            jnp.bfloat16)

        rdma_c = pltpu.make_async_remote_copy(
            src_ref=contrib_ref, dst_ref=recvc_ref,
            send_sem=send_sems.at[2], recv_sem=recv_sems.at[2],
            device_id=peer, device_id_type=_MESH)
        rdma_c.start()
        rdma_c.wait()

        out_ref[...] = out_ref[...] + recvc_ref[...].astype(jnp.float32)

    return pl.pallas_call(
        body,
        out_shape=jax.ShapeDtypeStruct((T, D), jnp.float32),
        in_specs=[pl.BlockSpec(memory_space=pltpu.VMEM)] * 4,
        out_specs=pl.BlockSpec(memory_space=pltpu.VMEM),
        scratch_shapes=[
            pltpu.VMEM((T, D), jnp.bfloat16),
            pltpu.VMEM((T, D), jnp.bfloat16),
            pltpu.VMEM((T, 1), jnp.int32),
            pltpu.VMEM((T, D), jnp.bfloat16),
            pltpu.VMEM((T, D), jnp.bfloat16),
            pltpu.SemaphoreType.DMA((3,)),
            pltpu.SemaphoreType.DMA((3,)),
        ],
        compiler_params=pltpu.CompilerParams(collective_id=0),
    )(x, assign2, W1, W2)


# baseline (device time: 96285 ns/iter reference)
import jax
import jax.numpy as jnp
from jax import lax
from jax.experimental import pallas as pl
from jax.experimental.pallas import tpu as pltpu

T = 1024
D = 1024
F = 2048
E_LOC = 2

_MESH = pl.DeviceIdType.MESH


def kernel(x, assign, W1, W2):
    assign2 = assign.reshape(T, 1)

    def body(x_ref, a_ref, w1_ref, w2_ref, out_ref,
             xb_ref, xpeer_ref, apeer_ref, contrib_ref, recvc_ref,
             send_sems, recv_sems):
        my_x = lax.axis_index("x")
        my_y = lax.axis_index("y")
        my_z = lax.axis_index("z")
        peer = (1 - my_x, my_y, my_z)

        barrier = pltpu.get_barrier_semaphore()
        pl.semaphore_signal(barrier, inc=1, device_id=peer,
                            device_id_type=_MESH)
        pl.semaphore_wait(barrier, 1)

        xb_ref[...] = x_ref[...].astype(jnp.bfloat16)

        rdma_x = pltpu.make_async_remote_copy(
            src_ref=xb_ref, dst_ref=xpeer_ref,
            send_sem=send_sems.at[0], recv_sem=recv_sems.at[0],
            device_id=peer, device_id_type=_MESH)
        rdma_x.start()
        rdma_a = pltpu.make_async_remote_copy(
            src_ref=a_ref, dst_ref=apeer_ref,
            send_sem=send_sems.at[1], recv_sem=recv_sems.at[1],
            device_id=peer, device_id_type=_MESH)
        rdma_a.start()

        def moe(xv, av):
            acc = None
            for k in range(E_LOC):
                ge = E_LOC * my_x + k
                xm = jnp.where(av == ge, xv, jnp.bfloat16(0.0))
                h = jnp.dot(xm, w1_ref[k].astype(jnp.bfloat16),
                            preferred_element_type=jnp.float32)
                h = jnp.maximum(h, 0.0).astype(jnp.bfloat16)
                o = jnp.dot(h, w2_ref[k].astype(jnp.bfloat16),
                            preferred_element_type=jnp.float32)
                acc = o if acc is None else acc + o
            return acc

        out_ref[...] = moe(xb_ref[...], a_ref[...])

        rdma_x.wait()
        rdma_a.wait()
        contrib_ref[...] = moe(xpeer_ref[...], apeer_ref[...]).astype(
            jnp.bfloat16)

        rdma_c = pltpu.make_async_remote_copy(
            src_ref=contrib_ref, dst_ref=recvc_ref,
            send_sem=send_sems.at[2], recv_sem=recv_sems.at[2],
            device_id=peer, device_id_type=_MESH)
        rdma_c.start()
        rdma_c.wait()

        out_ref[...] = out_ref[...] + recvc_ref[...].astype(jnp.float32)

    return pl.pallas_call(
        body,
        out_shape=jax.ShapeDtypeStruct((T, D), jnp.float32),
        in_specs=[pl.BlockSpec(memory_space=pltpu.VMEM)] * 4,
        out_specs=pl.BlockSpec(memory_space=pltpu.VMEM),
        scratch_shapes=[
            pltpu.VMEM((T, D), jnp.bfloat16),
            pltpu.VMEM((T, D), jnp.bfloat16),
            pltpu.VMEM((T, 1), jnp.int32),
            pltpu.VMEM((T, D), jnp.bfloat16),
            pltpu.VMEM((T, D), jnp.bfloat16),
            pltpu.SemaphoreType.DMA((3,)),
            pltpu.SemaphoreType.DMA((3,)),
        ],
        compiler_params=pltpu.CompilerParams(
            collective_id=0, vmem_limit_bytes=110 * 1024 * 1024),
    )(x, assign2, W1, W2)


# device time: 82944 ns/iter; 1.1608x vs baseline; 1.1608x over previous
import jax
import jax.numpy as jnp
from jax import lax
from jax.experimental import pallas as pl
from jax.experimental.pallas import tpu as pltpu

T = 1024
D = 1024
F = 2048
E_LOC = 2
N_CHUNK = 4

_MESH = pl.DeviceIdType.MESH


def kernel(x, assign, W1, W2):
    assign2 = assign.reshape(T, 1)

    def body(x_ref, a_ref, w1_ref, w2_ref, out_ref,
             xb_ref, xpeer_ref, apeer_ref, contrib_ref, recvc_ref,
             send_sems, recv_sems):
        my_x = lax.axis_index("x")
        my_y = lax.axis_index("y")
        my_z = lax.axis_index("z")
        peer = (1 - my_x, my_y, my_z)

        barrier = pltpu.get_barrier_semaphore()
        pl.semaphore_signal(barrier, inc=1, device_id=peer,
                            device_id_type=_MESH)
        pl.semaphore_wait(barrier, 1)

        xb_ref[...] = x_ref[...].astype(jnp.bfloat16)

        rdma_x = pltpu.make_async_remote_copy(
            src_ref=xb_ref, dst_ref=xpeer_ref,
            send_sem=send_sems.at[0], recv_sem=recv_sems.at[0],
            device_id=peer, device_id_type=_MESH)
        rdma_x.start()
        rdma_a = pltpu.make_async_remote_copy(
            src_ref=a_ref, dst_ref=apeer_ref,
            send_sem=send_sems.at[1], recv_sem=recv_sems.at[1],
            device_id=peer, device_id_type=_MESH)
        rdma_a.start()

        def moe(xv, av):
            acc = None
            for k in range(E_LOC):
                ge = E_LOC * my_x + k
                xm = jnp.where(av == ge, xv, jnp.bfloat16(0.0))
                h = jnp.dot(xm, w1_ref[k].astype(jnp.bfloat16),
                            preferred_element_type=jnp.float32)
                h = jnp.maximum(h, 0.0).astype(jnp.bfloat16)
                o = jnp.dot(h, w2_ref[k].astype(jnp.bfloat16),
                            preferred_element_type=jnp.float32)
                acc = o if acc is None else acc + o
            return acc

        out_ref[...] = moe(xb_ref[...], a_ref[...])

        rdma_x.wait()
        rdma_a.wait()
        rdma_c = []
        for c in range(N_CHUNK):
            sl = pl.ds(c * (T // N_CHUNK), T // N_CHUNK)
            contrib_ref[sl, :] = moe(
                xpeer_ref[sl, :], apeer_ref[sl, :]).astype(jnp.bfloat16)
            r = pltpu.make_async_remote_copy(
                src_ref=contrib_ref.at[sl, :], dst_ref=recvc_ref.at[sl, :],
                send_sem=send_sems.at[2 + c], recv_sem=recv_sems.at[2 + c],
                device_id=peer, device_id_type=_MESH)
            r.start()
            rdma_c.append(r)
        for r in rdma_c:
            r.wait()

        out_ref[...] = out_ref[...] + recvc_ref[...].astype(jnp.float32)

    return pl.pallas_call(
        body,
        out_shape=jax.ShapeDtypeStruct((T, D), jnp.float32),
        in_specs=[pl.BlockSpec(memory_space=pltpu.VMEM)] * 4,
        out_specs=pl.BlockSpec(memory_space=pltpu.VMEM),
        scratch_shapes=[
            pltpu.VMEM((T, D), jnp.bfloat16),
            pltpu.VMEM((T, D), jnp.bfloat16),
            pltpu.VMEM((T, 1), jnp.int32),
            pltpu.VMEM((T, D), jnp.bfloat16),
            pltpu.VMEM((T, D), jnp.bfloat16),
            pltpu.SemaphoreType.DMA((2 + N_CHUNK,)),
            pltpu.SemaphoreType.DMA((2 + N_CHUNK,)),
        ],
        compiler_params=pltpu.CompilerParams(
            collective_id=0, vmem_limit_bytes=110 * 1024 * 1024),
    )(x, assign2, W1, W2)
